# baseline (speedup 1.0000x reference)
import functools

import jax
import jax.numpy as jnp
from jax import lax
from jax.experimental import pallas as pl
from jax.experimental.pallas import tpu as pltpu
from jax.experimental.pallas import tpu_sc as plsc

NC = 2
NS = 16
NW = NC * NS
L = 16
CPN = 128


def _make_layer(in_f, out_f, chunk=128):
  npw = out_f // NW
  nchunks = npw // chunk
  assert out_f % (NW * chunk) == 0 and chunk % L == 0 and nchunks % 2 == 0
  mesh = plsc.VectorSubcoreMesh(core_axis_name="c", subcore_axis_name="s")

  @functools.partial(
      pl.kernel,
      mesh=mesh,
      out_type=jax.ShapeDtypeStruct((out_f,), jnp.float32),
      compiler_params=pltpu.CompilerParams(needs_layout_passes=False),
      scratch_types=[
          pltpu.VMEM((in_f,), jnp.float32),
          pltpu.VMEM((chunk * CPN // 2,), jnp.int32),
          pltpu.VMEM((chunk * CPN // 2,), jnp.int32),
          pltpu.VMEM((npw,), jnp.int32),
          pltpu.VMEM((npw,), jnp.float32),
          pltpu.VMEM((L * 17,), jnp.float32),
          pltpu.VMEM((L * 17,), jnp.float32),
          pltpu.SemaphoreType.DMA,
          pltpu.SemaphoreType.DMA,
      ],
  )
  def layer(x_hbm, conn_hbm, op_hbm, out_hbm,
            x_v, conn_a, conn_b, op_v, out_v, mn_buf, mx_buf, sem_a, sem_b):
    wid = lax.axis_index("s") * NC + lax.axis_index("c")
    row0 = wid * npw
    lanes17 = lax.iota(jnp.int32, L) * 17

    def conn_slice(c):
      return conn_hbm.at[
          pl.ds((row0 + c * chunk) * (CPN // 2), chunk * CPN // 2)]

    pltpu.async_copy(conn_slice(0), conn_a, sem_a)
    pltpu.sync_copy(x_hbm, x_v)
    pltpu.sync_copy(op_hbm.at[pl.ds(row0, npw)], op_v)

    def do_chunk(conn_v, ci):
      def block_body(nb, _):
        base = nb * (L * CPN // 2)
        for n in range(L):
          vs = []
          for jo in range(CPN // L // 2):
            w = conn_v[pl.ds(base + n * (CPN // 2) + jo * L, L)]
            clo = jnp.bitwise_and(w, 0xFFFF)
            chi = lax.shift_right_logical(w, 16)
            vs.append(plsc.load_gather(x_v, [clo]))
            vs.append(plsc.load_gather(x_v, [chi]))
          mns, mxs = list(vs), list(vs)
          while len(mns) > 1:
            mns = [jnp.minimum(a, b) for a, b in zip(mns[::2], mns[1::2])]
            mxs = [jnp.maximum(a, b) for a, b in zip(mxs[::2], mxs[1::2])]
          mn_buf[pl.ds(n * 17, L)] = mns[0]
          mx_buf[pl.ds(n * 17, L)] = mxs[0]
        mns = [plsc.load_gather(mn_buf, [lanes17 + k]) for k in range(L)]
        mxs = [plsc.load_gather(mx_buf, [lanes17 + k]) for k in range(L)]
        while len(mns) > 1:
          mns = [jnp.minimum(a, b) for a, b in zip(mns[::2], mns[1::2])]
          mxs = [jnp.maximum(a, b) for a, b in zip(mxs[::2], mxs[1::2])]
        o = ci * chunk + nb * L
        ops = op_v[pl.ds(o, L)]
        out_v[pl.ds(o, L)] = jnp.where(ops == 1, mxs[0], mns[0])
        return 0

      lax.fori_loop(0, chunk // L, block_body, 0)

    def pair_body(p, _):
      c0 = 2 * p
      pltpu.async_copy(conn_slice(c0 + 1), conn_b, sem_b)
      pltpu.make_async_copy(conn_slice(c0), conn_a, sem_a).wait()
      do_chunk(conn_a, c0)

      @pl.when(c0 + 2 < nchunks)
      def _():
        pltpu.async_copy(conn_slice(c0 + 2), conn_a, sem_a)

      pltpu.make_async_copy(conn_slice(c0 + 1), conn_b, sem_b).wait()
      do_chunk(conn_b, c0 + 1)
      return 0

    lax.fori_loop(0, nchunks // 2, pair_body, 0)
    pltpu.sync_copy(out_v, out_hbm.at[pl.ds(row0, npw)])

  return layer


_IN_F = 65536
_HID = [16384, 8192, 16384]
_SIZES = [_IN_F] + _HID + [_IN_F]
_LAYERS = [_make_layer(_SIZES[i], _SIZES[i + 1]) for i in range(4)]


def _pack16(conn):
  c16 = conn.astype(jnp.uint16).reshape(conn.shape[0], CPN // 2, 2)
  return lax.bitcast_convert_type(c16, jnp.int32).reshape(-1)


def kernel(x, conn0, op0, conn1, op1, conn2, op2, conn3, op3):
  h0 = _LAYERS[0](x, _pack16(conn0), op0)
  h1 = _LAYERS[1](h0, _pack16(conn1), op1)
  h2 = _LAYERS[2](h1, _pack16(conn2), op2)
  h3 = _LAYERS[3](h2, _pack16(conn3), op3)
  return h3, jnp.concatenate([h0, h1, h2], axis=0)


# speedup vs baseline: 3.3509x; 3.3509x over previous
import functools

import jax
import jax.numpy as jnp
from jax import lax
from jax.experimental import pallas as pl
from jax.experimental.pallas import tpu as pltpu
from jax.experimental.pallas import tpu_sc as plsc

NC = 2
NS = 16
NW = NC * NS
L = 16
CPN = 128
CHUNK = 128

_IN_F = 65536
_HID = [16384, 8192, 16384]
_SIZES = [_IN_F] + _HID + [_IN_F]

_MESH = plsc.VectorSubcoreMesh(core_axis_name="c", subcore_axis_name="s")


@functools.partial(
    pl.kernel,
    mesh=_MESH,
    out_type=[jax.ShapeDtypeStruct((_SIZES[i + 1],), jnp.float32)
              for i in range(4)] + [jax.ShapeDtypeStruct((2 * L,), jnp.int32)],
    compiler_params=pltpu.CompilerParams(needs_layout_passes=False),
    scratch_types=[
        pltpu.VMEM((_IN_F,), jnp.float32),
        pltpu.VMEM((CHUNK * CPN,), jnp.int32),
        pltpu.VMEM((CHUNK * CPN,), jnp.int32),
        pltpu.VMEM((_IN_F // NW,), jnp.int32),
        pltpu.VMEM((_IN_F // NW,), jnp.float32),
        pltpu.VMEM((L * 17,), jnp.float32),
        pltpu.VMEM((L * 17,), jnp.float32),
        pltpu.VMEM((L,), jnp.int32),
        pltpu.VMEM((L,), jnp.int32),
        pltpu.SemaphoreType.DMA,
        pltpu.SemaphoreType.DMA,
    ],
)
def _net(x_hbm, conn0, op0, conn1, op1, conn2, op2, conn3, op3,
         h0_hbm, h1_hbm, h2_hbm, h3_hbm, flags_hbm,
         x_v, conn_a, conn_b, op_v, out_v, mn_buf, mx_buf,
         flag_v, token_v, sem_a, sem_b):
  cid = lax.axis_index("c")
  sid = lax.axis_index("s")
  wid = sid * NC + cid
  lanes17 = lax.iota(jnp.int32, L) * 17

  @pl.when(sid == 0)
  def _():
    token_v[pl.ds(0, L)] = jnp.zeros((L,), jnp.int32)
    pltpu.sync_copy(token_v, flags_hbm.at[pl.ds(cid * L, L)])

  def run_layer(in_f, out_f, src_hbm, conn_hbm, op_hbm, out_hbm):
    npw = out_f // NW
    nchunks = npw // CHUNK
    row0 = wid * npw

    def conn_slice(c):
      return conn_hbm.at[pl.ds((row0 + c * CHUNK) * CPN, CHUNK * CPN)]

    pltpu.async_copy(conn_slice(0), conn_a, sem_a)
    pltpu.sync_copy(src_hbm, x_v.at[pl.ds(0, in_f)])
    pltpu.sync_copy(op_hbm.at[pl.ds(row0, npw)], op_v.at[pl.ds(0, npw)])

    def do_chunk(conn_v, ci):
      def block_body(nb, _):
        base = nb * (L * CPN)
        for n in range(L):
          vs = []
          for jo in range(CPN // L):
            c = conn_v[pl.ds(base + n * CPN + jo * L, L)]
            vs.append(plsc.load_gather(x_v, [c]))
          mns, mxs = list(vs), list(vs)
          while len(mns) > 1:
            mns = [jnp.minimum(a, b) for a, b in zip(mns[::2], mns[1::2])]
            mxs = [jnp.maximum(a, b) for a, b in zip(mxs[::2], mxs[1::2])]
          mn_buf[pl.ds(n * 17, L)] = mns[0]
          mx_buf[pl.ds(n * 17, L)] = mxs[0]
        mns = [plsc.load_gather(mn_buf, [lanes17 + k]) for k in range(L)]
        mxs = [plsc.load_gather(mx_buf, [lanes17 + k]) for k in range(L)]
        while len(mns) > 1:
          mns = [jnp.minimum(a, b) for a, b in zip(mns[::2], mns[1::2])]
          mxs = [jnp.maximum(a, b) for a, b in zip(mxs[::2], mxs[1::2])]
        o = ci * CHUNK + nb * L
        ops = op_v[pl.ds(o, L)]
        out_v[pl.ds(o, L)] = jnp.where(ops == 1, mxs[0], mns[0])
        return 0

      lax.fori_loop(0, CHUNK // L, block_body, 0)

    def pair_body(p, _):
      c0 = 2 * p
      pltpu.async_copy(conn_slice(c0 + 1), conn_b, sem_b)
      pltpu.make_async_copy(conn_slice(c0), conn_a, sem_a).wait()
      do_chunk(conn_a, c0)

      @pl.when(c0 + 2 < nchunks)
      def _():
        pltpu.async_copy(conn_slice(c0 + 2), conn_a, sem_a)

      pltpu.make_async_copy(conn_slice(c0 + 1), conn_b, sem_b).wait()
      do_chunk(conn_b, c0 + 1)
      return 0

    lax.fori_loop(0, nchunks // 2, pair_body, 0)
    pltpu.sync_copy(out_v.at[pl.ds(0, npw)], out_hbm.at[pl.ds(row0, npw)])

  def boundary(tok):
    plsc.subcore_barrier()

    @pl.when(sid == 0)
    def _():
      token_v[pl.ds(0, L)] = jnp.full((L,), tok, jnp.int32)
      pltpu.sync_copy(token_v, flags_hbm.at[pl.ds(cid * L, L)])

      def poll(done):
        pltpu.sync_copy(flags_hbm.at[pl.ds((1 - cid) * L, L)], flag_v)
        return jnp.all(flag_v[pl.ds(0, L)] == tok)

      lax.while_loop(lambda d: jnp.logical_not(d), poll, jnp.bool_(False))

    plsc.subcore_barrier()

  outs = [h0_hbm, h1_hbm, h2_hbm, h3_hbm]
  srcs = [x_hbm, h0_hbm, h1_hbm, h2_hbm]
  conns = [conn0, conn1, conn2, conn3]
  opss = [op0, op1, op2, op3]
  for li in range(4):
    run_layer(_SIZES[li], _SIZES[li + 1], srcs[li], conns[li], opss[li],
              outs[li])
    if li < 3:
      boundary(li + 1)


def kernel(x, conn0, op0, conn1, op1, conn2, op2, conn3, op3):
  h0, h1, h2, h3, _ = _net(x, conn0.reshape(-1), op0, conn1.reshape(-1), op1,
                           conn2.reshape(-1), op2, conn3.reshape(-1), op3)
  return h3, jnp.concatenate([h0, h1, h2], axis=0)


# speedup vs baseline: 3.5023x; 1.0452x over previous
import functools

import jax
import jax.numpy as jnp
from jax import lax
from jax.experimental import pallas as pl
from jax.experimental.pallas import tpu as pltpu
from jax.experimental.pallas import tpu_sc as plsc

NC = 2
NS = 16
NW = NC * NS
L = 16
CPN = 128
CHUNK = 128

_IN_F = 65536
_HID = [16384, 8192, 16384]
_SIZES = [_IN_F] + _HID + [_IN_F]

_MESH = plsc.VectorSubcoreMesh(core_axis_name="c", subcore_axis_name="s")


@functools.partial(
    pl.kernel,
    mesh=_MESH,
    out_type=[jax.ShapeDtypeStruct((_SIZES[i + 1],), jnp.float32)
              for i in range(4)] +
    [jax.ShapeDtypeStruct((sum(_HID),), jnp.float32),
     jax.ShapeDtypeStruct((2 * L,), jnp.int32)],
    compiler_params=pltpu.CompilerParams(needs_layout_passes=False),
    scratch_types=[
        pltpu.VMEM((_IN_F,), jnp.float32),
        pltpu.VMEM((CHUNK * CPN,), jnp.int32),
        pltpu.VMEM((CHUNK * CPN,), jnp.int32),
        pltpu.VMEM((_IN_F // NW,), jnp.int32),
        pltpu.VMEM((_IN_F // NW,), jnp.float32),
        pltpu.VMEM((L * 17,), jnp.float32),
        pltpu.VMEM((L * 17,), jnp.float32),
        pltpu.VMEM((L,), jnp.int32),
        pltpu.VMEM((L,), jnp.int32),
        pltpu.SemaphoreType.DMA,
        pltpu.SemaphoreType.DMA,
        pltpu.SemaphoreType.DMA,
    ],
)
def _net(x_hbm, conn0, op0, conn1, op1, conn2, op2, conn3, op3,
         h0_hbm, h1_hbm, h2_hbm, h3_hbm, acts_hbm, flags_hbm,
         x_v, conn_a, conn_b, op_v, out_v, mn_buf, mx_buf,
         flag_v, token_v, sem_a, sem_b, sem_op):
  cid = lax.axis_index("c")
  sid = lax.axis_index("s")
  wid = sid * NC + cid
  lanes17 = lax.iota(jnp.int32, L) * 17

  @pl.when(sid == 0)
  def _():
    token_v[pl.ds(0, L)] = jnp.zeros((L,), jnp.int32)
    pltpu.sync_copy(token_v, flags_hbm.at[pl.ds(cid * L, L)])

  def conn_slice_of(conn_hbm, out_f, c):
    row0 = wid * (out_f // NW)
    return conn_hbm.at[pl.ds((row0 + c * CHUNK) * CPN, CHUNK * CPN)]

  def prologue(out_f, conn_hbm, op_hbm):
    npw = out_f // NW
    row0 = wid * npw
    pltpu.async_copy(conn_slice_of(conn_hbm, out_f, 0), conn_a, sem_a)
    pltpu.async_copy(op_hbm.at[pl.ds(row0, npw)], op_v.at[pl.ds(0, npw)],
                     sem_op)

  def run_layer(in_f, out_f, src_hbm, conn_hbm, op_hbm, out_hbm, act_off):
    npw = out_f // NW
    nchunks = npw // CHUNK
    row0 = wid * npw

    def conn_slice(c):
      return conn_slice_of(conn_hbm, out_f, c)

    pltpu.sync_copy(src_hbm, x_v.at[pl.ds(0, in_f)])
    pltpu.make_async_copy(
        op_hbm.at[pl.ds(row0, npw)], op_v.at[pl.ds(0, npw)], sem_op).wait()

    def do_chunk(conn_v, ci):
      def block_body(nb, _):
        base = nb * (L * CPN)
        for n in range(L):
          vs = []
          for jo in range(CPN // L):
            c = conn_v[pl.ds(base + n * CPN + jo * L, L)]
            vs.append(plsc.load_gather(x_v, [c]))
          mns, mxs = list(vs), list(vs)
          while len(mns) > 1:
            mns = [jnp.minimum(a, b) for a, b in zip(mns[::2], mns[1::2])]
            mxs = [jnp.maximum(a, b) for a, b in zip(mxs[::2], mxs[1::2])]
          mn_buf[pl.ds(n * 17, L)] = mns[0]
          mx_buf[pl.ds(n * 17, L)] = mxs[0]
        mns = [plsc.load_gather(mn_buf, [lanes17 + k]) for k in range(L)]
        mxs = [plsc.load_gather(mx_buf, [lanes17 + k]) for k in range(L)]
        while len(mns) > 1:
          mns = [jnp.minimum(a, b) for a, b in zip(mns[::2], mns[1::2])]
          mxs = [jnp.maximum(a, b) for a, b in zip(mxs[::2], mxs[1::2])]
        o = ci * CHUNK + nb * L
        ops = op_v[pl.ds(o, L)]
        out_v[pl.ds(o, L)] = jnp.where(ops == 1, mxs[0], mns[0])
        return 0

      lax.fori_loop(0, CHUNK // L, block_body, 0)

    def pair_body(p, _):
      c0 = 2 * p
      pltpu.async_copy(conn_slice(c0 + 1), conn_b, sem_b)
      pltpu.make_async_copy(conn_slice(c0), conn_a, sem_a).wait()
      do_chunk(conn_a, c0)

      @pl.when(c0 + 2 < nchunks)
      def _():
        pltpu.async_copy(conn_slice(c0 + 2), conn_a, sem_a)

      pltpu.make_async_copy(conn_slice(c0 + 1), conn_b, sem_b).wait()
      do_chunk(conn_b, c0 + 1)
      return 0

    lax.fori_loop(0, nchunks // 2, pair_body, 0)
    pltpu.sync_copy(out_v.at[pl.ds(0, npw)], out_hbm.at[pl.ds(row0, npw)])
    if act_off is not None:
      pltpu.sync_copy(out_v.at[pl.ds(0, npw)],
                      acts_hbm.at[pl.ds(act_off + row0, npw)])

  def boundary(tok):
    plsc.subcore_barrier()

    @pl.when(sid == 0)
    def _():
      token_v[pl.ds(0, L)] = jnp.full((L,), tok, jnp.int32)
      pltpu.sync_copy(token_v, flags_hbm.at[pl.ds(cid * L, L)])

      def poll(done):
        pltpu.sync_copy(flags_hbm.at[pl.ds((1 - cid) * L, L)], flag_v)
        return jnp.all(flag_v[pl.ds(0, L)] == tok)

      lax.while_loop(lambda d: jnp.logical_not(d), poll, jnp.bool_(False))

    plsc.subcore_barrier()

  outs = [h0_hbm, h1_hbm, h2_hbm, h3_hbm]
  srcs = [x_hbm, h0_hbm, h1_hbm, h2_hbm]
  conns = [conn0, conn1, conn2, conn3]
  opss = [op0, op1, op2, op3]
  act_offs = [0, _HID[0], _HID[0] + _HID[1], None]
  prologue(_SIZES[1], conns[0], opss[0])
  for li in range(4):
    run_layer(_SIZES[li], _SIZES[li + 1], srcs[li], conns[li], opss[li],
              outs[li], act_offs[li])
    if li < 3:
      prologue(_SIZES[li + 2], conns[li + 1], opss[li + 1])
      boundary(li + 1)


def kernel(x, conn0, op0, conn1, op1, conn2, op2, conn3, op3):
  _, _, _, h3, acts, _ = _net(
      x, conn0.reshape(-1), op0, conn1.reshape(-1), op1,
      conn2.reshape(-1), op2, conn3.reshape(-1), op3)
  return h3, acts


# speedup vs baseline: 3.5450x; 1.0122x over previous
import functools

import jax
import jax.numpy as jnp
from jax import lax
from jax.experimental import pallas as pl
from jax.experimental.pallas import tpu as pltpu
from jax.experimental.pallas import tpu_sc as plsc

NC = 2
NS = 16
NW = NC * NS
L = 16
CPN = 128
CHUNK = 128

_IN_F = 65536
_HID = [16384, 8192, 16384]
_SIZES = [_IN_F] + _HID + [_IN_F]

_MESH = plsc.VectorSubcoreMesh(core_axis_name="c", subcore_axis_name="s")


@functools.partial(
    pl.kernel,
    mesh=_MESH,
    out_type=[jax.ShapeDtypeStruct((_SIZES[i + 1],), jnp.float32)
              for i in range(4)] +
    [jax.ShapeDtypeStruct((sum(_HID),), jnp.float32),
     jax.ShapeDtypeStruct((2 * L,), jnp.int32)],
    compiler_params=pltpu.CompilerParams(needs_layout_passes=False),
    scratch_types=[
        pltpu.VMEM((_IN_F,), jnp.float32),
        pltpu.VMEM((CHUNK * CPN,), jnp.int32),
        pltpu.VMEM((CHUNK * CPN,), jnp.int32),
        pltpu.VMEM((_IN_F // NW,), jnp.int32),
        pltpu.VMEM((_IN_F // NW,), jnp.float32),
        pltpu.VMEM((L * 17,), jnp.float32),
        pltpu.VMEM((L * 17,), jnp.float32),
        pltpu.VMEM((L,), jnp.int32),
        pltpu.VMEM((L,), jnp.int32),
        pltpu.SemaphoreType.DMA,
        pltpu.SemaphoreType.DMA,
        pltpu.SemaphoreType.DMA,
    ],
)
def _net(x_hbm, conn0, op0, conn1, op1, conn2, op2, conn3, op3,
         h0_hbm, h1_hbm, h2_hbm, h3_hbm, acts_hbm, flags_hbm,
         x_v, conn_a, conn_b, op_v, out_v, mn_buf, mx_buf,
         flag_v, token_v, sem_a, sem_b, sem_op):
  cid = lax.axis_index("c")
  sid = lax.axis_index("s")
  wid = sid * NC + cid
  lanes17 = lax.iota(jnp.int32, L) * 17

  @pl.when(sid == 0)
  def _():
    token_v[pl.ds(0, L)] = jnp.zeros((L,), jnp.int32)
    pltpu.sync_copy(token_v, flags_hbm.at[pl.ds(cid * L, L)])

  def conn_slice_of(conn_hbm, out_f, c):
    row0 = wid * (out_f // NW)
    return conn_hbm.at[pl.ds((row0 + c * CHUNK) * CPN, CHUNK * CPN)]

  def prologue(out_f, conn_hbm, op_hbm):
    npw = out_f // NW
    row0 = wid * npw
    pltpu.async_copy(conn_slice_of(conn_hbm, out_f, 0), conn_a, sem_a)
    pltpu.async_copy(op_hbm.at[pl.ds(row0, npw)], op_v.at[pl.ds(0, npw)],
                     sem_op)

  def run_layer(in_f, out_f, src_hbm, conn_hbm, op_hbm, out_hbm, act_off):
    npw = out_f // NW
    nchunks = npw // CHUNK
    row0 = wid * npw

    def conn_slice(c):
      return conn_slice_of(conn_hbm, out_f, c)

    pltpu.sync_copy(src_hbm, x_v.at[pl.ds(0, in_f)])
    pltpu.make_async_copy(
        op_hbm.at[pl.ds(row0, npw)], op_v.at[pl.ds(0, npw)], sem_op).wait()

    def do_chunk(conn_v, ci):
      def block_body(nb, _):
        base = nb * (L * CPN)
        o = ci * CHUNK + nb * L
        ops = op_v[pl.ds(o, L)]
        for n in range(L):
          vs = []
          for jo in range(CPN // L):
            c = conn_v[pl.ds(base + n * CPN + jo * L, L)]
            vs.append(plsc.load_gather(x_v, [c]))
          mns, mxs = list(vs), list(vs)
          while len(mns) > 1:
            mns = [jnp.minimum(a, b) for a, b in zip(mns[::2], mns[1::2])]
            mxs = [jnp.maximum(a, b) for a, b in zip(mxs[::2], mxs[1::2])]
          mn_buf[pl.ds(n * 17, L)] = jnp.where(ops[n] == 1, mxs[0], mns[0])
        opm = ops == 1
        cols = [plsc.load_gather(mn_buf, [lanes17 + k]) for k in range(L)]
        while len(cols) > 1:
          cols = [jnp.where(opm, jnp.maximum(a, b), jnp.minimum(a, b))
                  for a, b in zip(cols[::2], cols[1::2])]
        out_v[pl.ds(o, L)] = cols[0]
        return 0

      lax.fori_loop(0, CHUNK // L, block_body, 0)

    def pair_body(p, _):
      c0 = 2 * p
      pltpu.async_copy(conn_slice(c0 + 1), conn_b, sem_b)
      pltpu.make_async_copy(conn_slice(c0), conn_a, sem_a).wait()
      do_chunk(conn_a, c0)

      @pl.when(c0 + 2 < nchunks)
      def _():
        pltpu.async_copy(conn_slice(c0 + 2), conn_a, sem_a)

      pltpu.make_async_copy(conn_slice(c0 + 1), conn_b, sem_b).wait()
      do_chunk(conn_b, c0 + 1)
      return 0

    lax.fori_loop(0, nchunks // 2, pair_body, 0)
    pltpu.sync_copy(out_v.at[pl.ds(0, npw)], out_hbm.at[pl.ds(row0, npw)])
    if act_off is not None:
      pltpu.sync_copy(out_v.at[pl.ds(0, npw)],
                      acts_hbm.at[pl.ds(act_off + row0, npw)])

  def boundary(tok):
    plsc.subcore_barrier()

    @pl.when(sid == 0)
    def _():
      token_v[pl.ds(0, L)] = jnp.full((L,), tok, jnp.int32)
      pltpu.sync_copy(token_v, flags_hbm.at[pl.ds(cid * L, L)])

      def poll(done):
        pltpu.sync_copy(flags_hbm.at[pl.ds((1 - cid) * L, L)], flag_v)
        return jnp.all(flag_v[pl.ds(0, L)] == tok)

      lax.while_loop(lambda d: jnp.logical_not(d), poll, jnp.bool_(False))

    plsc.subcore_barrier()

  outs = [h0_hbm, h1_hbm, h2_hbm, h3_hbm]
  srcs = [x_hbm, h0_hbm, h1_hbm, h2_hbm]
  conns = [conn0, conn1, conn2, conn3]
  opss = [op0, op1, op2, op3]
  act_offs = [0, _HID[0], _HID[0] + _HID[1], None]
  prologue(_SIZES[1], conns[0], opss[0])
  for li in range(4):
    run_layer(_SIZES[li], _SIZES[li + 1], srcs[li], conns[li], opss[li],
              outs[li], act_offs[li])
    if li < 3:
      prologue(_SIZES[li + 2], conns[li + 1], opss[li + 1])
      boundary(li + 1)


def kernel(x, conn0, op0, conn1, op1, conn2, op2, conn3, op3):
  _, _, _, h3, acts, _ = _net(
      x, conn0.reshape(-1), op0, conn1.reshape(-1), op1,
      conn2.reshape(-1), op2, conn3.reshape(-1), op3)
  return h3, acts
